# baseline (speedup 1.0000x reference)
import functools

import jax
import jax.numpy as jnp
from jax.experimental import pallas as pl

N = 10000
F = 128
ROW_BLK = 1000
GRID = N // ROW_BLK


def _pre_body(x_ref, w1_ref, w2_ref, bpre_ref, a_ref, b_ref, b2_ref):
    x = x_ref[...]
    a_ref[...] = jnp.dot(x, w1_ref[...], preferred_element_type=jnp.float32) + bpre_ref[...]
    b = jnp.dot(x, w2_ref[...], preferred_element_type=jnp.float32)
    b_ref[...] = b
    b2_ref[...] = b * b


def _avg_body(degp_ref, out_ref):
    out_ref[...] = jnp.sum(jnp.log(degp_ref[...] + 1.0), axis=None, keepdims=True) / N


def _post_body(x_ref, a_ref, s_ref, q_ref, mx_ref, mn_ref, deg_ref, avg_ref,
               wpost_ref, bpost_ref, wlin_ref, blin_ref, out_ref, st_ref):
    deg = deg_ref[...]
    degc = jnp.maximum(deg, 1.0)
    a = a_ref[...]
    s = s_ref[...]
    mean = (deg * a + s) / degc
    s2 = deg * a * a + 2.0 * a * s + q_ref[...]
    std = jnp.sqrt(jnp.maximum(s2 / degc - mean * mean, 0.0) + 1e-5)
    pos = deg > 0.0
    mx = jnp.where(pos, a + mx_ref[...], 0.0)
    mn = jnp.where(pos, a + mn_ref[...], 0.0)
    log_deg = jnp.log(degc + 1.0)
    avg_log = avg_ref[0, 0]
    amp = log_deg / avg_log
    att = avg_log / log_deg
    cat = jnp.concatenate(
        [x_ref[...], mean, mn, mx, std,
         amp * mean, amp * mn, amp * mx, amp * std,
         att * mean, att * mn, att * mx, att * std], axis=1)
    out = jnp.dot(cat, wpost_ref[...], preferred_element_type=jnp.float32) + bpost_ref[...]
    out = jnp.dot(out, wlin_ref[...], preferred_element_type=jnp.float32) + blin_ref[...]
    out_ref[...] = out

    @pl.when(pl.program_id(0) == 0)
    def _():
        st_ref[...] = jnp.zeros_like(st_ref)

    st_ref[0:1, :] += jnp.sum(out, axis=0, keepdims=True)
    st_ref[1:2, :] += jnp.sum(out * out, axis=0, keepdims=True)


def _h1_body(out_ref, x_ref, st_ref, g_ref, b_ref, h1_ref, st1_ref):
    mu = st_ref[0:1, :] / N
    var = st_ref[1:2, :] / N - mu * mu
    inv = jax.lax.rsqrt(var + 1e-5)
    h1 = jnp.maximum((out_ref[...] - mu) * inv * g_ref[...] + b_ref[...], 0.0) + x_ref[...]
    h1_ref[...] = h1

    @pl.when(pl.program_id(0) == 0)
    def _():
        st1_ref[...] = jnp.zeros_like(st1_ref)

    st1_ref[0:1, :] += jnp.sum(h1, axis=0, keepdims=True)
    st1_ref[1:2, :] += jnp.sum(h1 * h1, axis=0, keepdims=True)


def _ffn_body(h1_ref, st1_ref, g1_ref, b1_ref, wf1_ref, bf1_ref, wf2_ref, bf2_ref,
              t_ref, st2_ref):
    mu = st1_ref[0:1, :] / N
    var = st1_ref[1:2, :] / N - mu * mu
    inv = jax.lax.rsqrt(var + 1e-5)
    h1 = h1_ref[...]
    h2 = (h1 - mu) * inv * g1_ref[...] + b1_ref[...]
    h2 = jnp.maximum(jnp.dot(h2, wf1_ref[...], preferred_element_type=jnp.float32) + bf1_ref[...], 0.0)
    h2 = jnp.dot(h2, wf2_ref[...], preferred_element_type=jnp.float32) + bf2_ref[...]
    t = h1 + h2
    t_ref[...] = t

    @pl.when(pl.program_id(0) == 0)
    def _():
        st2_ref[...] = jnp.zeros_like(st2_ref)

    st2_ref[0:1, :] += jnp.sum(t, axis=0, keepdims=True)
    st2_ref[1:2, :] += jnp.sum(t * t, axis=0, keepdims=True)


def _fin_body(t_ref, st2_ref, g2_ref, b2_ref, y_ref):
    mu = st2_ref[0:1, :] / N
    var = st2_ref[1:2, :] / N - mu * mu
    inv = jax.lax.rsqrt(var + 1e-5)
    y_ref[...] = (t_ref[...] - mu) * inv * g2_ref[...] + b2_ref[...]


def _row_spec():
    return pl.BlockSpec((ROW_BLK, F), lambda i: (i, 0))


def _full_spec(shape):
    return pl.BlockSpec(shape, lambda i: tuple(0 for _ in shape))


def _segment_stats(b_nodes, b2_nodes, src, dst):
    rows = jnp.take(b_nodes, src, axis=0)
    rows2 = jnp.take(b2_nodes, src, axis=0)
    s = jax.ops.segment_sum(rows, dst, num_segments=N)
    q = jax.ops.segment_sum(rows2, dst, num_segments=N)
    mx = jax.ops.segment_max(rows, dst, num_segments=N)
    mn = -jax.ops.segment_max(-rows, dst, num_segments=N)
    deg = jnp.bincount(dst, length=N).astype(jnp.float32)
    return s, q, mx, mn, deg


def kernel(x, edge_index, W_pre, b_pre, W_post, b_post, W_lin, b_lin, bn_g, bn_b,
           W_ff1, b_ff1, W_ff2, b_ff2, bn1_g, bn1_b, bn2_g, bn2_b):
    f32 = jnp.float32
    src = edge_index[0]
    dst = edge_index[1]

    a_nodes, b_nodes, b2_nodes = pl.pallas_call(
        _pre_body,
        grid=(GRID,),
        in_specs=[_row_spec(), _full_spec((F, F)), _full_spec((F, F)),
                  _full_spec((1, F))],
        out_specs=[_row_spec(), _row_spec(), _row_spec()],
        out_shape=[jax.ShapeDtypeStruct((N, F), f32)] * 3,
    )(x, W_pre[:F], W_pre[F:], b_pre.reshape(1, F))

    s, q, mx, mn, deg = _segment_stats(b_nodes, b2_nodes, src, dst)

    deg_pad = jnp.pad(deg, (0, 10240 - N)).reshape(80, 128)
    avg_log = pl.pallas_call(
        _avg_body,
        out_shape=jax.ShapeDtypeStruct((1, 1), f32),
    )(deg_pad)

    deg2 = deg.reshape(N, 1)
    out2, st0 = pl.pallas_call(
        _post_body,
        grid=(GRID,),
        in_specs=[_row_spec(), _row_spec(), _row_spec(), _row_spec(),
                  _row_spec(), _row_spec(),
                  pl.BlockSpec((ROW_BLK, 1), lambda i: (i, 0)),
                  _full_spec((1, 1)),
                  _full_spec((13 * F, F)), _full_spec((1, F)),
                  _full_spec((F, F)), _full_spec((1, F))],
        out_specs=[_row_spec(), _full_spec((8, F))],
        out_shape=[jax.ShapeDtypeStruct((N, F), f32),
                   jax.ShapeDtypeStruct((8, F), f32)],
    )(x, a_nodes, s, q, mx, mn, deg2, avg_log,
      W_post, b_post.reshape(1, F), W_lin, b_lin.reshape(1, F))

    h1, st1 = pl.pallas_call(
        _h1_body,
        grid=(GRID,),
        in_specs=[_row_spec(), _row_spec(), _full_spec((8, F)),
                  _full_spec((1, F)), _full_spec((1, F))],
        out_specs=[_row_spec(), _full_spec((8, F))],
        out_shape=[jax.ShapeDtypeStruct((N, F), f32),
                   jax.ShapeDtypeStruct((8, F), f32)],
    )(out2, x, st0, bn_g.reshape(1, F), bn_b.reshape(1, F))

    t, st2 = pl.pallas_call(
        _ffn_body,
        grid=(GRID,),
        in_specs=[_row_spec(), _full_spec((8, F)), _full_spec((1, F)),
                  _full_spec((1, F)), _full_spec((F, 2 * F)),
                  _full_spec((1, 2 * F)), _full_spec((2 * F, F)),
                  _full_spec((1, F))],
        out_specs=[_row_spec(), _full_spec((8, F))],
        out_shape=[jax.ShapeDtypeStruct((N, F), f32),
                   jax.ShapeDtypeStruct((8, F), f32)],
    )(h1, st1, bn1_g.reshape(1, F), bn1_b.reshape(1, F),
      W_ff1, b_ff1.reshape(1, 2 * F), W_ff2, b_ff2.reshape(1, F))

    y = pl.pallas_call(
        _fin_body,
        grid=(GRID,),
        in_specs=[_row_spec(), _full_spec((8, F)), _full_spec((1, F)),
                  _full_spec((1, F))],
        out_specs=_row_spec(),
        out_shape=jax.ShapeDtypeStruct((N, F), f32),
    )(t, st2, bn2_g.reshape(1, F), bn2_b.reshape(1, F))

    return y
